# Initial kernel scaffold; baseline (speedup 1.0000x reference)
import functools

import jax
import jax.numpy as jnp
from jax import lax
from jax.experimental import pallas as pl
from jax.experimental.pallas import tpu as pltpu
from jax.experimental.pallas import tpu_sc as plsc

Q = 1024
D = 64
K = 100000
TOPK = 3
THRESHOLD = 0.0

BK = 2048
NB = (K + BK - 1) // BK
BIG = jnp.int32(2 ** 30)

B = Q * TOPK
SC_CORES = 2
SC_SUBCORES = 16
NW = SC_CORES * SC_SUBCORES
BPW = B // NW


def _normalize(x):
    return x / (jnp.sqrt(jnp.sum(x * x, axis=1, keepdims=True)) + 1e-12)


def _row_topk_kernel(q_ref, k_ref, idx_ref, val_scr):
    step = pl.program_id(0)

    @pl.when(step == 0)
    def _init():
        val_scr[...] = jnp.full((Q, TOPK), -jnp.inf, jnp.float32)
        idx_ref[...] = jnp.full((Q, TOPK), BIG, jnp.int32)

    qn = _normalize(q_ref[...])
    kn = _normalize(k_ref[...])
    s = lax.dot_general(qn, kn, (((1,), (1,)), ((), ())),
                        preferred_element_type=jnp.float32)
    gidx = lax.broadcasted_iota(jnp.int32, (Q, BK), 1) + step * BK
    s = jnp.where(gidx < K, s, -jnp.inf)

    loc_v, loc_i = [], []
    for r in range(TOPK):
        m = jnp.max(s, axis=1, keepdims=True)
        i = jnp.min(jnp.where(s == m, gidx, BIG), axis=1, keepdims=True)
        loc_v.append(m)
        loc_i.append(i)
        if r < TOPK - 1:
            s = jnp.where(gidx == i, -jnp.inf, s)

    v6 = jnp.concatenate([val_scr[...]] + loc_v, axis=1)
    i6 = jnp.concatenate([idx_ref[...]] + loc_i, axis=1)
    new_v, new_i = [], []
    for r in range(TOPK):
        m = jnp.max(v6, axis=1, keepdims=True)
        i = jnp.min(jnp.where(v6 == m, i6, BIG), axis=1, keepdims=True)
        new_v.append(m)
        new_i.append(i)
        if r < TOPK - 1:
            v6 = jnp.where(i6 == i, -jnp.inf, v6)
    val_scr[...] = jnp.concatenate(new_v, axis=1)
    idx_ref[...] = jnp.concatenate(new_i, axis=1)


def _row_topk(queries, keys):
    return pl.pallas_call(
        _row_topk_kernel,
        grid=(NB,),
        in_specs=[
            pl.BlockSpec((Q, D), lambda k: (0, 0)),
            pl.BlockSpec((BK, D), lambda k: (k, 0)),
        ],
        out_specs=pl.BlockSpec((Q, TOPK), lambda k: (0, 0)),
        out_shape=jax.ShapeDtypeStruct((Q, TOPK), jnp.int32),
        scratch_shapes=[pltpu.VMEM((Q, TOPK), jnp.float32)],
        compiler_params=pltpu.CompilerParams(
            dimension_semantics=("arbitrary",)),
    )(queries, keys)


_sc_mesh = plsc.VectorSubcoreMesh(core_axis_name="c", subcore_axis_name="s")


@functools.partial(
    pl.kernel,
    mesh=_sc_mesh,
    out_type=jax.ShapeDtypeStruct((B, D), jnp.float32),
    scratch_types=[
        pltpu.VMEM((BPW,), jnp.int32),
        pltpu.VMEM((BPW, D), jnp.float32),
        pltpu.SemaphoreType.DMA,
    ],
)
def _gather_rows(keys_hbm, idx_hbm, out_hbm, idx_v, rows_v, sem):
    wid = lax.axis_index("s") * SC_CORES + lax.axis_index("c")
    base = wid * BPW
    pltpu.sync_copy(idx_hbm.at[pl.ds(base, BPW)], idx_v)
    pltpu.async_copy(keys_hbm.at[idx_v], rows_v, sem).wait()
    pltpu.sync_copy(rows_v, out_hbm.at[pl.ds(base, BPW)])


def _mutual_kernel(q_ref, sk_ref, out_ref):
    qn = _normalize(q_ref[...])
    skn = _normalize(sk_ref[...])
    s = lax.dot_general(qn, skn, (((1,), (1,)), ((), ())),
                        preferred_element_type=jnp.float32)
    ridx = lax.broadcasted_iota(jnp.int32, (Q, B), 0)
    cidx = lax.broadcasted_iota(jnp.int32, (Q, B), 1)
    owner = cidx // TOPK
    v = jnp.max(jnp.where(ridx == owner, s, -jnp.inf), axis=0, keepdims=True)
    precede = (s > v) | ((s == v) & (ridx < owner))
    cnt = jnp.sum(precede.astype(jnp.float32), axis=0, keepdims=True)
    keep = (cnt < float(TOPK)) & (v > THRESHOLD)
    out_ref[...] = jnp.broadcast_to(jnp.where(keep, v, 0.0), (8, B))


def _mutual(queries, sel_keys):
    return pl.pallas_call(
        _mutual_kernel,
        in_specs=[
            pl.BlockSpec((Q, D), lambda: (0, 0)),
            pl.BlockSpec((B, D), lambda: (0, 0)),
        ],
        out_specs=pl.BlockSpec((8, B), lambda: (0, 0)),
        out_shape=jax.ShapeDtypeStruct((8, B), jnp.float32),
    )(queries, sel_keys)


def kernel(queries, keys):
    row_idx = _row_topk(queries, keys)
    sel = _gather_rows(keys, row_idx.reshape(B))
    out = _mutual(queries, sel)
    return out[0].reshape(Q, TOPK)


# speedup vs baseline: 43.3564x; 43.3564x over previous
import functools

import jax
import jax.numpy as jnp
from jax import lax
from jax.experimental import pallas as pl
from jax.experimental.pallas import tpu as pltpu
from jax.experimental.pallas import tpu_sc as plsc

Q = 1024
D = 64
K = 100000
TOPK = 3
THRESHOLD = 0.0

BK = 2048
NB = (K + BK - 1) // BK
BIG = 2 ** 30

B = Q * TOPK
SC_CORES = 2
SC_SUBCORES = 16
NW = SC_CORES * SC_SUBCORES
BPW = B // NW


def _normalize(x):
    return x / (jnp.sqrt(jnp.sum(x * x, axis=1, keepdims=True)) + 1e-12)


def _row_topk_kernel(q_ref, k_ref, idx_ref, val_scr):
    step = pl.program_id(0)

    @pl.when(step == 0)
    def _init():
        val_scr[...] = jnp.full((Q, TOPK), -jnp.inf, jnp.float32)
        idx_ref[...] = jnp.full((Q, TOPK), BIG, jnp.int32)

    qn = _normalize(q_ref[...])
    kn = _normalize(k_ref[...])
    s = lax.dot_general(qn, kn, (((1,), (1,)), ((), ())),
                        preferred_element_type=jnp.float32)
    gidx = lax.broadcasted_iota(jnp.int32, (Q, BK), 1) + step * BK
    s = jnp.where(gidx < K, s, -jnp.inf)

    loc_v, loc_i = [], []
    for r in range(TOPK):
        m = jnp.max(s, axis=1, keepdims=True)
        i = jnp.min(jnp.where(s == m, gidx, BIG), axis=1, keepdims=True)
        loc_v.append(m)
        loc_i.append(i)
        if r < TOPK - 1:
            s = jnp.where(gidx == i, -jnp.inf, s)

    v6 = jnp.concatenate([val_scr[...]] + loc_v, axis=1)
    i6 = jnp.concatenate([idx_ref[...]] + loc_i, axis=1)
    new_v, new_i = [], []
    for r in range(TOPK):
        m = jnp.max(v6, axis=1, keepdims=True)
        i = jnp.min(jnp.where(v6 == m, i6, BIG), axis=1, keepdims=True)
        new_v.append(m)
        new_i.append(i)
        if r < TOPK - 1:
            v6 = jnp.where(i6 == i, -jnp.inf, v6)
    val_scr[...] = jnp.concatenate(new_v, axis=1)
    idx_ref[...] = jnp.concatenate(new_i, axis=1)


def _row_topk(queries, keys):
    return pl.pallas_call(
        _row_topk_kernel,
        grid=(NB,),
        in_specs=[
            pl.BlockSpec((Q, D), lambda k: (0, 0)),
            pl.BlockSpec((BK, D), lambda k: (k, 0)),
        ],
        out_specs=pl.BlockSpec((Q, TOPK), lambda k: (0, 0)),
        out_shape=jax.ShapeDtypeStruct((Q, TOPK), jnp.int32),
        scratch_shapes=[pltpu.VMEM((Q, TOPK), jnp.float32)],
        compiler_params=pltpu.CompilerParams(
            dimension_semantics=("arbitrary",)),
    )(queries, keys)


D2 = 2 * D


@functools.lru_cache(maxsize=None)
def _build_gather_rows():
    mesh = plsc.VectorSubcoreMesh(core_axis_name="c", subcore_axis_name="s")

    @functools.partial(
        pl.kernel,
        mesh=mesh,
        out_type=jax.ShapeDtypeStruct((B, D2), jnp.float32),
        scratch_types=[
            pltpu.VMEM((BPW,), jnp.int32),
            pltpu.VMEM((BPW,), jnp.int32),
            pltpu.VMEM((BPW, D2), jnp.float32),
            pltpu.SemaphoreType.DMA,
        ],
    )
    def _gather_rows(keys2_hbm, idx_hbm, out_hbm, idx_v, idx2_v, rows_v, sem):
        wid = lax.axis_index("s") * SC_CORES + lax.axis_index("c")
        base = wid * BPW
        pltpu.sync_copy(idx_hbm.at[pl.ds(base, BPW)], idx_v)
        for c in range(BPW // 16):
            sl = pl.ds(c * 16, 16)
            idx2_v[sl] = lax.shift_right_logical(idx_v[sl], 1)
        pltpu.async_copy(keys2_hbm.at[idx2_v], rows_v, sem).wait()
        pltpu.sync_copy(rows_v, out_hbm.at[pl.ds(base, BPW)])

    return _gather_rows


def _mutual_kernel(q_ref, sk_ref, idx_ref, out_ref):
    qn = _normalize(q_ref[...])
    sk = sk_ref[...]
    even = _normalize(sk[:, :D])
    odd = _normalize(sk[:, D:])
    skn = jnp.concatenate([even, odd], axis=1)
    par = idx_ref[...] & 1
    lane = lax.broadcasted_iota(jnp.int32, (B, D2), 1)
    skn = jnp.where((lane >= D) == (par == 1), skn, 0.0)
    qn2 = jnp.concatenate([qn, qn], axis=1)
    s = lax.dot_general(qn2, skn, (((1,), (1,)), ((), ())),
                        preferred_element_type=jnp.float32)
    ridx = lax.broadcasted_iota(jnp.int32, (Q, B), 0)
    cidx = lax.broadcasted_iota(jnp.int32, (Q, B), 1)
    owner = cidx // TOPK
    v = jnp.max(jnp.where(ridx == owner, s, -jnp.inf), axis=0, keepdims=True)
    precede = (s > v) | ((s == v) & (ridx < owner))
    cnt = jnp.sum(precede.astype(jnp.float32), axis=0, keepdims=True)
    keep = (cnt < float(TOPK)) & (v > THRESHOLD)
    out_ref[...] = jnp.broadcast_to(jnp.where(keep, v, 0.0), (8, B))


def _mutual(queries, sel_keys, idx_col):
    return pl.pallas_call(
        _mutual_kernel,
        in_specs=[
            pl.BlockSpec((Q, D), lambda: (0, 0)),
            pl.BlockSpec((B, D2), lambda: (0, 0)),
            pl.BlockSpec((B, 1), lambda: (0, 0)),
        ],
        out_specs=pl.BlockSpec((8, B), lambda: (0, 0)),
        out_shape=jax.ShapeDtypeStruct((8, B), jnp.float32),
    )(queries, sel_keys, idx_col)


def kernel(queries, keys):
    row_idx = _row_topk(queries, keys)
    keys2 = keys.reshape(K // 2, D2)
    idx_flat = row_idx.reshape(B)
    sel = _build_gather_rows()(keys2, idx_flat)
    out = _mutual(queries, sel, row_idx.reshape(B, 1))
    return out[0].reshape(Q, TOPK)
